# baseline (speedup 1.0000x reference)
import jax
import jax.numpy as jnp
from jax.experimental import pallas as pl
from jax.experimental.pallas import tpu as pltpu

HEIGHT = 32
WIDTH = 32
D_MODEL = 768


H_BLK = 8


def _add_pos_body(x_ref, row_ref, col_ref, out_ref):
    x = x_ref[0]
    pos = row_ref[...][:, None, :] + col_ref[...][None, :, :]
    out_ref[0] = x + pos


def kernel(x, row_embed, col_embed):
    batch, seq_len, d = x.shape
    x4 = x.reshape(batch, HEIGHT, WIDTH, d)
    out = pl.pallas_call(
        _add_pos_body,
        grid=(batch, HEIGHT // H_BLK),
        in_specs=[
            pl.BlockSpec((1, H_BLK, WIDTH, d), lambda b, h: (b, h, 0, 0)),
            pl.BlockSpec((H_BLK, d), lambda b, h: (h, 0)),
            pl.BlockSpec((WIDTH, d), lambda b, h: (0, 0)),
        ],
        out_specs=pl.BlockSpec((1, H_BLK, WIDTH, d), lambda b, h: (b, h, 0, 0)),
        out_shape=jax.ShapeDtypeStruct((batch, HEIGHT, WIDTH, d), x.dtype),
    )(x4, row_embed, col_embed)
    return out.reshape(batch, seq_len, d)


# speedup vs baseline: 1.7813x; 1.7813x over previous
import jax
import jax.numpy as jnp
from jax.experimental import pallas as pl
from jax.experimental.pallas import tpu as pltpu

HEIGHT = 32
WIDTH = 32
D_MODEL = 768


B_BLK = 2


def _add_pos_body(x_ref, row_ref, col_ref, out_ref):
    pos = row_ref[...][None, :, None, :] + col_ref[...][None, None, :, :]
    out_ref[...] = x_ref[...] + pos


def kernel(x, row_embed, col_embed):
    batch, seq_len, d = x.shape
    x4 = x.reshape(batch, HEIGHT, WIDTH, d)
    out = pl.pallas_call(
        _add_pos_body,
        grid=(batch // B_BLK,),
        in_specs=[
            pl.BlockSpec((B_BLK, HEIGHT, WIDTH, d), lambda b: (b, 0, 0, 0)),
            pl.BlockSpec((HEIGHT, d), lambda b: (0, 0)),
            pl.BlockSpec((WIDTH, d), lambda b: (0, 0)),
        ],
        out_specs=pl.BlockSpec((B_BLK, HEIGHT, WIDTH, d), lambda b: (b, 0, 0, 0)),
        out_shape=jax.ShapeDtypeStruct((batch, HEIGHT, WIDTH, d), x.dtype),
    )(x4, row_embed, col_embed)
    return out.reshape(batch, seq_len, d)


# speedup vs baseline: 1.8018x; 1.0115x over previous
import jax
import jax.numpy as jnp
from jax.experimental import pallas as pl
from jax.experimental.pallas import tpu as pltpu

HEIGHT = 32
WIDTH = 32
D_MODEL = 768


B_BLK = 4


def _add_pos_body(x_ref, row_ref, col_ref, out_ref):
    pos = row_ref[...][None, :, None, :] + col_ref[...][None, None, :, :]
    out_ref[...] = x_ref[...] + pos


def kernel(x, row_embed, col_embed):
    batch, seq_len, d = x.shape
    x4 = x.reshape(batch, HEIGHT, WIDTH, d)
    out = pl.pallas_call(
        _add_pos_body,
        grid=(batch // B_BLK,),
        in_specs=[
            pl.BlockSpec((B_BLK, HEIGHT, WIDTH, d), lambda b: (b, 0, 0, 0)),
            pl.BlockSpec((HEIGHT, d), lambda b: (0, 0)),
            pl.BlockSpec((WIDTH, d), lambda b: (0, 0)),
        ],
        out_specs=pl.BlockSpec((B_BLK, HEIGHT, WIDTH, d), lambda b: (b, 0, 0, 0)),
        out_shape=jax.ShapeDtypeStruct((batch, HEIGHT, WIDTH, d), x.dtype),
    )(x4, row_embed, col_embed)
    return out.reshape(batch, seq_len, d)
